# baseline (speedup 1.0000x reference)
import jax
import jax.numpy as jnp
from jax.experimental import pallas as pl

EPG = 2


def _moe_body(eids_ref, x_ref, gw_ref, uw_ref, dw_ref, out_ref):
    g = pl.program_id(0)
    x = x_ref[...]
    acc = None
    for j in range(EPG):
        e = g * EPG + j
        gate = jax.lax.dot_general(x, gw_ref[j], (((1,), (1,)), ((), ())),
                                   preferred_element_type=jnp.float32)
        up = jax.lax.dot_general(x, uw_ref[j], (((1,), (1,)), ((), ())),
                                 preferred_element_type=jnp.float32)
        h = gate * jax.nn.sigmoid(gate) * up
        outp = jax.lax.dot_general(h, dw_ref[j], (((1,), (1,)), ((), ())),
                                   preferred_element_type=jnp.float32)
        contrib = jnp.where(eids_ref[...] == e, outp, 0.0)
        acc = contrib if acc is None else acc + contrib

    @pl.when(g == 0)
    def _():
        out_ref[...] = acc

    @pl.when(g > 0)
    def _():
        out_ref[...] += acc


def kernel(x, expert_ids, gate_weight, up_weight, down_weight):
    n, d = x.shape
    num_e, hidden, _ = gate_weight.shape
    eids = expert_ids.reshape(n, 1)
    return pl.pallas_call(
        _moe_body,
        grid=(num_e // EPG,),
        in_specs=[
            pl.BlockSpec((n, 1), lambda g: (0, 0)),
            pl.BlockSpec((n, d), lambda g: (0, 0)),
            pl.BlockSpec((EPG, hidden, d), lambda g: (g, 0, 0)),
            pl.BlockSpec((EPG, hidden, d), lambda g: (g, 0, 0)),
            pl.BlockSpec((EPG, d, hidden), lambda g: (g, 0, 0)),
        ],
        out_specs=pl.BlockSpec((n, d), lambda g: (0, 0)),
        out_shape=jax.ShapeDtypeStruct((n, d), jnp.float32),
    )(eids, x, gate_weight, up_weight, down_weight)
